# baseline (speedup 1.0000x reference)
import functools
import math

import jax
import jax.numpy as jnp
from jax import lax
from jax.experimental import pallas as pl
from jax.experimental.pallas import tpu as pltpu
from jax.experimental.pallas import tpu_sc as plsc

N = 10000
K = 32
E = N * K
D_NODE = 128
D_EDGE = 16
D_TIME = 128
Q_DIM = 256
H = 2
D_H = 128

NW = 32
EW = E // NW
SUP = 400
G = 80
NSUP = EW // SUP
NG = SUP // G

TN = 200
GRID = N // TN


def _sc_gather_fn():
    mesh = plsc.VectorSubcoreMesh(core_axis_name="c", subcore_axis_name="s")

    @functools.partial(
        pl.kernel,
        mesh=mesh,
        compiler_params=pltpu.CompilerParams(needs_layout_passes=False, use_tc_tiling_on_sc=False),
        out_type=[
            jax.ShapeDtypeStruct((E, D_NODE // 2), jnp.int32),
            jax.ShapeDtypeStruct((E,), jnp.float32),
        ],
        scratch_types=[
            pltpu.VMEM((N,), jnp.float32),
            pltpu.VMEM((SUP,), jnp.int32),
            pltpu.VMEM((SUP,), jnp.float32),
            pltpu.VMEM((SUP,), jnp.float32),
            pltpu.VMEM((SUP, D_NODE // 2), jnp.int32),
            pltpu.SemaphoreType.DMA,
            pltpu.SemaphoreType.DMA,
        ],
    )
    def sc_gather(emb_hbm, lu_hbm, src_hbm, ts_hbm, rows_out, dt_out,
                  lu_v, idx_v, ts_v, dt_v, rows_v, sem_s, sem_g):
        wid = lax.axis_index("s") * 2 + lax.axis_index("c")
        base = wid * EW
        pltpu.sync_copy(lu_hbm, lu_v)

        def step(s, carry):
            e0 = base + s * SUP
            cp_i = pltpu.async_copy(src_hbm.at[pl.ds(e0, SUP)], idx_v, sem_s)
            cp_t = pltpu.async_copy(ts_hbm.at[pl.ds(e0, SUP)], ts_v, sem_s)
            cp_i.wait()
            cp_t.wait()
            copies = [
                pltpu.async_copy(
                    emb_hbm.at[idx_v.at[pl.ds(g * G, G)]],
                    rows_v.at[pl.ds(g * G, G)],
                    sem_g,
                )
                for g in range(NG)
            ]
            for g in range(SUP // 16):
                iv = idx_v[pl.ds(g * 16, 16)]
                lu_g = plsc.load_gather(lu_v, [iv])
                dt_v[pl.ds(g * 16, 16)] = ts_v[pl.ds(g * 16, 16)] - lu_g
            for cp in copies:
                cp.wait()
            pltpu.sync_copy(rows_v, rows_out.at[pl.ds(e0, SUP)])
            pltpu.sync_copy(dt_v, dt_out.at[pl.ds(e0, SUP)])
            return carry

        lax.fori_loop(0, NSUP, step, 0)

    return sc_gather


_INV_2PI = 1.0 / (2.0 * math.pi)
_COS_C = (1.0, -19.739206314086914, 64.93917083740234, -85.45116424560547,
          60.176231384277344, -26.000532150268555, 6.575617790222168)


def _fast_cos(x):
    u = x * _INV_2PI
    r = u - jnp.round(u)
    s = r * r
    p = _COS_C[6]
    for c in (_COS_C[5], _COS_C[4], _COS_C[3], _COS_C[2], _COS_C[1],
              _COS_C[0]):
        p = p * s + c
    return p


def _tc_body(se_ref, dt_ref, ef_ref, emb_ref, tw_ref, tb_ref,
             wqe_ref, wqt_ref, bq_ref, wkp_ref,
             wvet_ref, wvf_ref, bv_ref, wo_ref, bo_ref,
             w1e_ref, w1h_ref, b1_ref, w2_ref, b2_ref, out_ref):
    f32 = jnp.float32
    dotT = lambda a, b: lax.dot_general(a, b, (((1,), (1,)), ((), ())),
                                        preferred_element_type=f32)
    dotN = lambda a, b: lax.dot_general(a, b, (((1,), (0,)), ((), ())),
                                        preferred_element_type=f32)

    emb_t = emb_ref[...]
    tw = tw_ref[...]
    tb = tb_ref[...]
    cosb = jnp.cos(tb)

    q = dotT(emb_t, wqe_ref[...]) + dotT(cosb, wqt_ref[...]) + bq_ref[...]

    dt = dt_ref[...]
    te = _fast_cos(dt[:, :, None] * tw[0][None, None, :]
                   + tb[0][None, None, :])
    se = se_ref[...].astype(jnp.float32)
    ef = ef_ref[...]

    wkp = wkp_ref[...]
    wvet = wvet_ref[...]
    wvf = wvf_ref[...]
    bv = bv_ref[...]

    scale = 1.0 / math.sqrt(float(D_H))
    ctxs = []
    for h in range(H):
        r0, r1 = h * D_H, (h + 1) * D_H
        qh = q[:, r0:r1]
        p = dotN(qh, wkp[r0:r1, :])
        pe = p[:, :D_NODE]
        pt = p[:, D_NODE:2 * D_NODE]
        pf = p[:, 2 * D_NODE:]
        s = (jnp.sum(se * pe[:, None, :] + te * pt[:, None, :], axis=2)
             + jnp.sum(ef * pf[:, None, :], axis=2))
        logits = s * scale
        m = jnp.max(logits, axis=1, keepdims=True)
        ex = jnp.exp(logits - m)
        att = ex / jnp.sum(ex, axis=1, keepdims=True)
        embbar = jnp.sum(att[:, :, None] * se, axis=1)
        efbar = jnp.sum(att[:, :, None] * ef, axis=1)
        tebar = jnp.sum(att[:, :, None] * te, axis=1)
        etbar = jnp.concatenate([embbar, tebar], axis=1)
        ctx_h = (dotT(etbar, wvet[r0:r1, :])
                 + dotT(efbar, wvf[r0:r1, :])
                 + bv[:, r0:r1])
        ctxs.append(ctx_h)

    ctx = jnp.concatenate(ctxs, axis=1)
    hb = dotT(ctx, wo_ref[...]) + bo_ref[...]
    h1 = jnp.maximum(dotT(emb_t, w1e_ref[...])
                     + dotT(hb, w1h_ref[...]) + b1_ref[...], 0.0)
    out_ref[...] = dotT(h1, w2_ref[...]) + b2_ref[...]


def _tc_attn(src_emb, dt, edge_feat, emb, time_w, time_b,
             Wq, bq, Wk, bk, Wv, bv, Wo, bo, W1, b1, W2, b2,
             interpret=False):
    se3 = src_emb.reshape(N, K, D_NODE)
    dt2 = dt.reshape(N, K)
    ef3 = edge_feat.reshape(N, K, D_EDGE)
    row = lambda v: v.reshape(1, -1)
    full2 = lambda a: pl.BlockSpec(a.shape, lambda i: (0, 0))

    args = (
        se3, dt2, ef3, emb,
        row(time_w), row(time_b),
        Wq[:, :D_NODE], Wq[:, D_NODE:], row(bq),
        jnp.concatenate([Wk[:, :D_NODE], Wk[:, D_NODE + D_EDGE:],
                         Wk[:, D_NODE:D_NODE + D_EDGE]], axis=1),
        jnp.concatenate([Wv[:, :D_NODE], Wv[:, D_NODE + D_EDGE:]], axis=1),
        Wv[:, D_NODE:D_NODE + D_EDGE],
        row(bv),
        Wo, row(bo),
        W1[:, :D_NODE], W1[:, D_NODE:], row(b1),
        W2, row(b2),
    )
    in_specs = [
        pl.BlockSpec((TN, K, D_NODE), lambda i: (i, 0, 0)),
        pl.BlockSpec((TN, K), lambda i: (i, 0)),
        pl.BlockSpec((TN, K, D_EDGE), lambda i: (i, 0, 0)),
        pl.BlockSpec((TN, D_NODE), lambda i: (i, 0)),
    ] + [full2(a) for a in args[4:]]

    return pl.pallas_call(
        _tc_body,
        grid=(GRID,),
        in_specs=in_specs,
        out_specs=pl.BlockSpec((TN, D_NODE), lambda i: (i, 0)),
        out_shape=jax.ShapeDtypeStruct((N, D_NODE), jnp.float32),
        interpret=interpret,
    )(*args)


def kernel(emb, edge_feat, timestamp, last_update, src_idx, time_w, time_b,
           Wq, bq, Wk, bk, Wv, bv, Wo, bo, W1, b1, W2, b2):
    emb_pack = lax.bitcast_convert_type(
        emb.astype(jnp.bfloat16).reshape(N, D_NODE // 2, 2), jnp.int32)
    rows_pack, dt = _sc_gather_fn()(emb_pack, last_update,
                                    src_idx.astype(jnp.int32), timestamp)
    src_emb = lax.bitcast_convert_type(
        rows_pack, jnp.bfloat16).reshape(E, D_NODE)
    return _tc_attn(src_emb, dt, edge_feat, emb, time_w, time_b,
                    Wq, bq, Wk, bk, Wv, bv, Wo, bo, W1, b1, W2, b2)


# speedup vs baseline: 2.7827x; 2.7827x over previous
import functools
import math

import jax
import jax.numpy as jnp
from jax import lax
from jax.experimental import pallas as pl
from jax.experimental.pallas import tpu as pltpu
from jax.experimental.pallas import tpu_sc as plsc

N = 10000
K = 32
E = N * K
D_NODE = 128
D_EDGE = 16
D_TIME = 128
Q_DIM = 256
H = 2
D_H = 128

NW = 32
EW = E // NW
SUP = 400
G = 80
NSUP = EW // SUP
NG = SUP // G

TN = 400
GRID = N // TN


def _sc_gather_fn():
    mesh = plsc.VectorSubcoreMesh(core_axis_name="c", subcore_axis_name="s")

    @functools.partial(
        pl.kernel,
        mesh=mesh,
        compiler_params=pltpu.CompilerParams(needs_layout_passes=False),
        out_type=[
            jax.ShapeDtypeStruct((E, D_NODE), jnp.float32),
            jax.ShapeDtypeStruct((E,), jnp.float32),
        ],
        scratch_types=[
            pltpu.VMEM((N,), jnp.float32),
            pltpu.VMEM((SUP,), jnp.int32),
            pltpu.VMEM((SUP,), jnp.float32),
            pltpu.VMEM((SUP,), jnp.float32),
            pltpu.VMEM((SUP, D_NODE), jnp.float32),
            pltpu.SemaphoreType.DMA,
            pltpu.SemaphoreType.DMA,
        ],
    )
    def sc_gather(emb_hbm, lu_hbm, src_hbm, ts_hbm, rows_out, dt_out,
                  lu_v, idx_v, ts_v, dt_v, rows_v, sem_s, sem_g):
        wid = lax.axis_index("s") * 2 + lax.axis_index("c")
        base = wid * EW
        pltpu.sync_copy(lu_hbm, lu_v)

        def step(s, carry):
            e0 = base + s * SUP
            cp_i = pltpu.async_copy(src_hbm.at[pl.ds(e0, SUP)], idx_v, sem_s)
            cp_t = pltpu.async_copy(ts_hbm.at[pl.ds(e0, SUP)], ts_v, sem_s)
            cp_i.wait()
            cp_t.wait()
            copies = [
                pltpu.async_copy(
                    emb_hbm.at[idx_v.at[pl.ds(g * G, G)]],
                    rows_v.at[pl.ds(g * G, G)],
                    sem_g,
                )
                for g in range(NG)
            ]
            for g in range(SUP // 16):
                iv = idx_v[pl.ds(g * 16, 16)]
                lu_g = plsc.load_gather(lu_v, [iv])
                dt_v[pl.ds(g * 16, 16)] = ts_v[pl.ds(g * 16, 16)] - lu_g
            for cp in copies:
                cp.wait()
            pltpu.sync_copy(rows_v, rows_out.at[pl.ds(e0, SUP)])
            pltpu.sync_copy(dt_v, dt_out.at[pl.ds(e0, SUP)])
            return carry

        lax.fori_loop(0, NSUP, step, 0)

    return sc_gather


_INV_2PI = 1.0 / (2.0 * math.pi)
_COS_C = (1.0, -19.739206314086914, 64.93917083740234, -85.45116424560547,
          60.176231384277344, -26.000532150268555, 6.575617790222168)


def _fast_cos(x):
    u = x * _INV_2PI
    r = u - jnp.round(u)
    s = r * r
    p = _COS_C[6]
    for c in (_COS_C[5], _COS_C[4], _COS_C[3], _COS_C[2], _COS_C[1],
              _COS_C[0]):
        p = p * s + c
    return p


def _tc_body(se_ref, dt_ref, ef_ref, emb_ref, tw_ref, tb_ref,
             wqe_ref, wqt_ref, bq_ref, wkp_ref,
             wvet_ref, wvf_ref, bv_ref, wo_ref, bo_ref,
             w1e_ref, w1h_ref, b1_ref, w2_ref, b2_ref, out_ref):
    f32 = jnp.float32
    dotT = lambda a, b: lax.dot_general(a, b, (((1,), (1,)), ((), ())),
                                        preferred_element_type=f32)
    dotN = lambda a, b: lax.dot_general(a, b, (((1,), (0,)), ((), ())),
                                        preferred_element_type=f32)

    emb_t = emb_ref[...]
    tw = tw_ref[...]
    tb = tb_ref[...]
    cosb = jnp.cos(tb)

    q = dotT(emb_t, wqe_ref[...]) + dotT(cosb, wqt_ref[...]) + bq_ref[...]

    dt = dt_ref[...]
    te = _fast_cos(dt[:, :, None] * tw[0][None, None, :]
                   + tb[0][None, None, :])
    se = se_ref[...]
    ef = ef_ref[...]

    wkp = wkp_ref[...]
    wvet = wvet_ref[...]
    wvf = wvf_ref[...]
    bv = bv_ref[...]

    scale = 1.0 / math.sqrt(float(D_H))
    ctxs = []
    for h in range(H):
        r0, r1 = h * D_H, (h + 1) * D_H
        qh = q[:, r0:r1]
        p = dotN(qh, wkp[r0:r1, :])
        pe = p[:, :D_NODE]
        pt = p[:, D_NODE:2 * D_NODE]
        pf = p[:, 2 * D_NODE:]
        s = (jnp.sum(se * pe[:, None, :] + te * pt[:, None, :], axis=2)
             + jnp.sum(ef * pf[:, None, :], axis=2))
        logits = s * scale
        m = jnp.max(logits, axis=1, keepdims=True)
        ex = jnp.exp(logits - m)
        att = ex / jnp.sum(ex, axis=1, keepdims=True)
        embbar = jnp.sum(att[:, :, None] * se, axis=1)
        efbar = jnp.sum(att[:, :, None] * ef, axis=1)
        tebar = jnp.sum(att[:, :, None] * te, axis=1)
        etbar = jnp.concatenate([embbar, tebar], axis=1)
        ctx_h = (dotT(etbar, wvet[r0:r1, :])
                 + dotT(efbar, wvf[r0:r1, :])
                 + bv[:, r0:r1])
        ctxs.append(ctx_h)

    ctx = jnp.concatenate(ctxs, axis=1)
    hb = dotT(ctx, wo_ref[...]) + bo_ref[...]
    h1 = jnp.maximum(dotT(emb_t, w1e_ref[...])
                     + dotT(hb, w1h_ref[...]) + b1_ref[...], 0.0)
    out_ref[...] = dotT(h1, w2_ref[...]) + b2_ref[...]


def _tc_attn(src_emb, dt, edge_feat, emb, time_w, time_b,
             Wq, bq, Wk, bk, Wv, bv, Wo, bo, W1, b1, W2, b2,
             interpret=False):
    se3 = src_emb.reshape(N, K, D_NODE)
    dt2 = dt.reshape(N, K)
    ef3 = edge_feat.reshape(N, K, D_EDGE)
    row = lambda v: v.reshape(1, -1)
    full2 = lambda a: pl.BlockSpec(a.shape, lambda i: (0, 0))

    args = (
        se3, dt2, ef3, emb,
        row(time_w), row(time_b),
        Wq[:, :D_NODE], Wq[:, D_NODE:], row(bq),
        jnp.concatenate([Wk[:, :D_NODE], Wk[:, D_NODE + D_EDGE:],
                         Wk[:, D_NODE:D_NODE + D_EDGE]], axis=1),
        jnp.concatenate([Wv[:, :D_NODE], Wv[:, D_NODE + D_EDGE:]], axis=1),
        Wv[:, D_NODE:D_NODE + D_EDGE],
        row(bv),
        Wo, row(bo),
        W1[:, :D_NODE], W1[:, D_NODE:], row(b1),
        W2, row(b2),
    )
    in_specs = [
        pl.BlockSpec((TN, K, D_NODE), lambda i: (i, 0, 0)),
        pl.BlockSpec((TN, K), lambda i: (i, 0)),
        pl.BlockSpec((TN, K, D_EDGE), lambda i: (i, 0, 0)),
        pl.BlockSpec((TN, D_NODE), lambda i: (i, 0)),
    ] + [full2(a) for a in args[4:]]

    return pl.pallas_call(
        _tc_body,
        grid=(GRID,),
        in_specs=in_specs,
        out_specs=pl.BlockSpec((TN, D_NODE), lambda i: (i, 0)),
        out_shape=jax.ShapeDtypeStruct((N, D_NODE), jnp.float32),
        interpret=interpret,
    )(*args)


def kernel(emb, edge_feat, timestamp, last_update, src_idx, time_w, time_b,
           Wq, bq, Wk, bk, Wv, bv, Wo, bo, W1, b1, W2, b2):
    src_emb, dt = _sc_gather_fn()(emb, last_update,
                                  src_idx.astype(jnp.int32), timestamp)
    return _tc_attn(src_emb, dt, edge_feat, emb, time_w, time_b,
                    Wq, bq, Wk, bk, Wv, bv, Wo, bo, W1, b1, W2, b2)
